# Initial kernel scaffold; baseline (speedup 1.0000x reference)
import jax
import jax.numpy as jnp
from jax.experimental import pallas as pl


def kernel(edge_index, batch, W0, W1, W2, b0, b1, b2):
    raise NotImplementedError("write your pallas kernel here")


# speedup vs baseline: 356.8955x; 356.8955x over previous
import functools

import jax
import jax.numpy as jnp
from jax import lax
from jax.experimental import pallas as pl
from jax.experimental.pallas import tpu as pltpu
from jax.experimental.pallas import tpu_sc as plsc

_N = 100000
_E = 6400000
_CLS = 10
_G = 64
_LANES = 128
_N_ROWS = 784
_N_PAD = _N_ROWS * _LANES
_NW = 32
_EPW = _E // _NW
_CHUNK = 20000
_NCHUNK = _EPW // _CHUNK
_ZC = _N_PAD // 16


def _sc_body(src_hbm, dst_hbm, g_hbm, z_hbm, out_hbm,
             sidx, didx, vals, g_sh, acc_sh):
    cid = lax.axis_index("c")
    sid = lax.axis_index("s")
    wid = sid * 2 + cid
    noff = sid * _ZC
    pltpu.sync_copy(z_hbm.at[pl.ds(noff, _ZC)], acc_sh.at[pl.ds(noff, _ZC)])
    pltpu.sync_copy(g_hbm.at[pl.ds(noff, _ZC)], g_sh.at[pl.ds(noff, _ZC)])
    plsc.subcore_barrier()

    def chunk(i, carry):
        base = pl.multiple_of(wid * _EPW + i * _CHUNK, 8)
        pltpu.sync_copy(src_hbm.at[pl.ds(base, _CHUNK)], sidx)
        pltpu.sync_copy(dst_hbm.at[pl.ds(base, _CHUNK)], didx)
        pltpu.sync_copy(g_sh.at[sidx], vals)
        pltpu.sync_copy(vals, acc_sh.at[didx], add=True)
        return carry

    lax.fori_loop(0, _NCHUNK, chunk, 0)

    plsc.subcore_barrier()
    pltpu.sync_copy(acc_sh.at[pl.ds(noff, _ZC)],
                    out_hbm.at[cid, pl.ds(noff, _ZC)])


_sc_pass = functools.partial(
    pl.kernel,
    mesh=plsc.VectorSubcoreMesh(core_axis_name="c", subcore_axis_name="s"),
    out_type=jax.ShapeDtypeStruct((2, _N_PAD), jnp.float32),
    scratch_types=[
        pltpu.VMEM((_CHUNK,), jnp.int32),
        pltpu.VMEM((_CHUNK,), jnp.int32),
        pltpu.VMEM((_CHUNK,), jnp.float32),
        pltpu.VMEM_SHARED((_N_PAD,), jnp.float32),
        pltpu.VMEM_SHARED((_N_PAD,), jnp.float32),
    ],
)(_sc_body)


def _graph_sums(b, h):
    return jnp.stack(
        [jnp.sum(jnp.where(b == g, h, 0.0), axis=0) for g in range(_G)], axis=0)


def _tc_dinv_body(p0_ref, p1_ref, batch_ref, dinv_ref, counts_ref):
    deg = p0_ref[...] + p1_ref[...] + 1.0
    dinv_ref[...] = lax.rsqrt(deg)
    b = batch_ref[...]
    counts_ref[...] = _graph_sums(b, jnp.ones_like(deg))


_tc_dinv = pl.pallas_call(
    _tc_dinv_body,
    out_shape=(jax.ShapeDtypeStruct((_N_ROWS, _LANES), jnp.float32),
               jax.ShapeDtypeStruct((_G, _LANES), jnp.float32)),
)


def _tc_combine_body(p0_ref, p1_ref, gprev_ref, dinv_ref, batch_ref,
                     gnew_ref, sums_ref):
    d = dinv_ref[...]
    h = d * (p0_ref[...] + p1_ref[...] + gprev_ref[...])
    gnew_ref[...] = d * h
    sums_ref[...] = _graph_sums(batch_ref[...], h)


_tc_combine = pl.pallas_call(
    _tc_combine_body,
    out_shape=(jax.ShapeDtypeStruct((_N_ROWS, _LANES), jnp.float32),
               jax.ShapeDtypeStruct((_G, _LANES), jnp.float32)),
)


def _tc_final_body(s1_ref, s2_ref, s3_ref, counts_ref, w_ref, b_ref, out_ref):
    cnt = jnp.maximum(jnp.sum(counts_ref[...], axis=1), 1.0)
    logits = jnp.zeros((_G, _LANES), jnp.float32)
    for i, s_ref in enumerate((s1_ref, s2_ref, s3_ref)):
        pooled = jnp.sum(s_ref[...], axis=1) / cnt
        logits = logits + pooled[:, None] * w_ref[i, :][None, :]
    logits = logits + jnp.sum(b_ref[...], axis=0)[None, :]
    lane = lax.broadcasted_iota(jnp.int32, (_G, _LANES), 1)
    masked = jnp.where(lane < _CLS, logits, -1e30)
    m = jnp.max(masked, axis=1, keepdims=True)
    ex = jnp.where(lane < _CLS, jnp.exp(masked - m), 0.0)
    lse = jnp.log(jnp.sum(ex, axis=1, keepdims=True)) + m
    out_ref[...] = logits - lse


_tc_final = pl.pallas_call(
    _tc_final_body,
    out_shape=jax.ShapeDtypeStruct((_G, _LANES), jnp.float32),
)


def kernel(edge_index, batch, W0, W1, W2, b0, b1, b2):
    src1d = edge_index[0]
    dst1d = edge_index[1]
    batch2d = jnp.pad(batch, (0, _N_PAD - _N),
                      constant_values=_G).reshape(_N_ROWS, _LANES)
    zeros_n = jnp.zeros((_N_PAD,), jnp.float32)
    ones_n = jnp.ones((_N_PAD,), jnp.float32)

    degp = _sc_pass(dst1d, dst1d, ones_n, zeros_n)
    dinv2d, counts = _tc_dinv(degp[0].reshape(_N_ROWS, _LANES),
                              degp[1].reshape(_N_ROWS, _LANES), batch2d)

    g2d = dinv2d
    sums = []
    for _ in range(3):
        pp = _sc_pass(src1d, dst1d, g2d.reshape(_N_PAD), zeros_n)
        g2d, s = _tc_combine(pp[0].reshape(_N_ROWS, _LANES),
                             pp[1].reshape(_N_ROWS, _LANES),
                             g2d, dinv2d, batch2d)
        sums.append(s)

    w_pad = jnp.zeros((8, _LANES), jnp.float32)
    w_pad = w_pad.at[:3, :_CLS].set(jnp.concatenate([W0, W1, W2], axis=0))
    b_pad = jnp.zeros((8, _LANES), jnp.float32)
    b_pad = b_pad.at[:3, :_CLS].set(jnp.stack([b0, b1, b2], axis=0))

    out = _tc_final(sums[0], sums[1], sums[2], counts, w_pad, b_pad)
    return out[:, :_CLS]


# speedup vs baseline: 451.9724x; 1.2664x over previous
import functools

import jax
import jax.numpy as jnp
from jax import lax
from jax.experimental import pallas as pl
from jax.experimental.pallas import tpu as pltpu
from jax.experimental.pallas import tpu_sc as plsc

_N = 100000
_E = 6400000
_CLS = 10
_G = 64
_LANES = 128
_N_ROWS = 784
_N_PAD = _N_ROWS * _LANES
_NW = 32
_EPW = _E // _NW
_CHUNK = 20000
_NCHUNK = _EPW // _CHUNK
_ZC = _N_PAD // 16


_MESH = plsc.VectorSubcoreMesh(core_axis_name="c", subcore_axis_name="s")


def _sc_prop_body(src_hbm, dst_hbm, g_hbm, z_hbm, out_hbm,
                  sidx0, sidx1, didx0, didx1, vals, g_sh, acc_sh,
                  sem_s0, sem_s1, sem_d0, sem_d1):
    cid = lax.axis_index("c")
    sid = lax.axis_index("s")
    wid = sid * 2 + cid
    noff = sid * _ZC
    pltpu.sync_copy(z_hbm.at[pl.ds(noff, _ZC)], acc_sh.at[pl.ds(noff, _ZC)])
    pltpu.sync_copy(g_hbm.at[pl.ds(noff, _ZC)], g_sh.at[pl.ds(noff, _ZC)])
    plsc.subcore_barrier()

    sbufs, dbufs = (sidx0, sidx1), (didx0, didx1)
    ssems, dsems = (sem_s0, sem_s1), (sem_d0, sem_d1)
    base0 = wid * _EPW

    def start(i):
        b = i % 2
        return (pltpu.async_copy(src_hbm.at[pl.ds(base0 + i * _CHUNK, _CHUNK)],
                                 sbufs[b], ssems[b]),
                pltpu.async_copy(dst_hbm.at[pl.ds(base0 + i * _CHUNK, _CHUNK)],
                                 dbufs[b], dsems[b]))

    pend = start(0)
    for i in range(_NCHUNK):
        b = i % 2
        pend[0].wait()
        pend[1].wait()
        if i + 1 < _NCHUNK:
            pend = start(i + 1)
        pltpu.sync_copy(g_sh.at[sbufs[b]], vals)
        pltpu.sync_copy(vals, acc_sh.at[dbufs[b]], add=True)

    plsc.subcore_barrier()
    pltpu.sync_copy(acc_sh.at[pl.ds(noff, _ZC)],
                    out_hbm.at[cid, pl.ds(noff, _ZC)])


_sc_pass = functools.partial(
    pl.kernel,
    mesh=_MESH,
    out_type=jax.ShapeDtypeStruct((2, _N_PAD), jnp.float32),
    scratch_types=[
        pltpu.VMEM((_CHUNK,), jnp.int32),
        pltpu.VMEM((_CHUNK,), jnp.int32),
        pltpu.VMEM((_CHUNK,), jnp.int32),
        pltpu.VMEM((_CHUNK,), jnp.int32),
        pltpu.VMEM((_CHUNK,), jnp.float32),
        pltpu.VMEM_SHARED((_N_PAD,), jnp.float32),
        pltpu.VMEM_SHARED((_N_PAD,), jnp.float32),
        pltpu.SemaphoreType.DMA,
        pltpu.SemaphoreType.DMA,
        pltpu.SemaphoreType.DMA,
        pltpu.SemaphoreType.DMA,
    ],
)(_sc_prop_body)


def _sc_count_body(dst_hbm, ones_hbm, z_hbm, out_hbm,
                   didx0, didx1, ones_v, acc_sh, sem_d0, sem_d1):
    cid = lax.axis_index("c")
    sid = lax.axis_index("s")
    wid = sid * 2 + cid
    noff = sid * _ZC
    pltpu.sync_copy(z_hbm.at[pl.ds(noff, _ZC)], acc_sh.at[pl.ds(noff, _ZC)])
    pltpu.sync_copy(ones_hbm.at[pl.ds(0, _CHUNK)], ones_v)
    plsc.subcore_barrier()

    dbufs, dsems = (didx0, didx1), (sem_d0, sem_d1)
    base0 = wid * _EPW

    def start(i):
        b = i % 2
        return pltpu.async_copy(dst_hbm.at[pl.ds(base0 + i * _CHUNK, _CHUNK)],
                                dbufs[b], dsems[b])

    pend = start(0)
    for i in range(_NCHUNK):
        b = i % 2
        pend.wait()
        if i + 1 < _NCHUNK:
            pend = start(i + 1)
        pltpu.sync_copy(ones_v, acc_sh.at[dbufs[b]], add=True)

    plsc.subcore_barrier()
    pltpu.sync_copy(acc_sh.at[pl.ds(noff, _ZC)],
                    out_hbm.at[cid, pl.ds(noff, _ZC)])


_sc_count = functools.partial(
    pl.kernel,
    mesh=_MESH,
    out_type=jax.ShapeDtypeStruct((2, _N_PAD), jnp.float32),
    scratch_types=[
        pltpu.VMEM((_CHUNK,), jnp.int32),
        pltpu.VMEM((_CHUNK,), jnp.int32),
        pltpu.VMEM((_CHUNK,), jnp.float32),
        pltpu.VMEM_SHARED((_N_PAD,), jnp.float32),
        pltpu.SemaphoreType.DMA,
        pltpu.SemaphoreType.DMA,
    ],
)(_sc_count_body)


def _graph_sums(b, h):
    return jnp.stack(
        [jnp.sum(jnp.where(b == g, h, 0.0), axis=0) for g in range(_G)], axis=0)


def _tc_dinv_body(p0_ref, p1_ref, batch_ref, dinv_ref, counts_ref):
    deg = p0_ref[...] + p1_ref[...] + 1.0
    dinv_ref[...] = lax.rsqrt(deg)
    b = batch_ref[...]
    counts_ref[...] = _graph_sums(b, jnp.ones_like(deg))


_tc_dinv = pl.pallas_call(
    _tc_dinv_body,
    out_shape=(jax.ShapeDtypeStruct((_N_ROWS, _LANES), jnp.float32),
               jax.ShapeDtypeStruct((_G, _LANES), jnp.float32)),
)


def _tc_combine_body(p0_ref, p1_ref, gprev_ref, dinv_ref, batch_ref,
                     gnew_ref, sums_ref):
    d = dinv_ref[...]
    h = d * (p0_ref[...] + p1_ref[...] + gprev_ref[...])
    gnew_ref[...] = d * h
    sums_ref[...] = _graph_sums(batch_ref[...], h)


_tc_combine = pl.pallas_call(
    _tc_combine_body,
    out_shape=(jax.ShapeDtypeStruct((_N_ROWS, _LANES), jnp.float32),
               jax.ShapeDtypeStruct((_G, _LANES), jnp.float32)),
)


def _tc_final_body(s1_ref, s2_ref, s3_ref, counts_ref, w_ref, b_ref, out_ref):
    cnt = jnp.maximum(jnp.sum(counts_ref[...], axis=1), 1.0)
    logits = jnp.zeros((_G, _LANES), jnp.float32)
    for i, s_ref in enumerate((s1_ref, s2_ref, s3_ref)):
        pooled = jnp.sum(s_ref[...], axis=1) / cnt
        logits = logits + pooled[:, None] * w_ref[i, :][None, :]
    logits = logits + jnp.sum(b_ref[...], axis=0)[None, :]
    lane = lax.broadcasted_iota(jnp.int32, (_G, _LANES), 1)
    masked = jnp.where(lane < _CLS, logits, -1e30)
    m = jnp.max(masked, axis=1, keepdims=True)
    ex = jnp.where(lane < _CLS, jnp.exp(masked - m), 0.0)
    lse = jnp.log(jnp.sum(ex, axis=1, keepdims=True)) + m
    out_ref[...] = logits - lse


_tc_final = pl.pallas_call(
    _tc_final_body,
    out_shape=jax.ShapeDtypeStruct((_G, _LANES), jnp.float32),
)


def kernel(edge_index, batch, W0, W1, W2, b0, b1, b2):
    src1d = edge_index[0]
    dst1d = edge_index[1]
    batch2d = jnp.pad(batch, (0, _N_PAD - _N),
                      constant_values=_G).reshape(_N_ROWS, _LANES)
    zeros_n = jnp.zeros((_N_PAD,), jnp.float32)
    ones_n = jnp.ones((_N_PAD,), jnp.float32)

    degp = _sc_count(dst1d, ones_n, zeros_n)
    dinv2d, counts = _tc_dinv(degp[0].reshape(_N_ROWS, _LANES),
                              degp[1].reshape(_N_ROWS, _LANES), batch2d)

    g2d = dinv2d
    sums = []
    for _ in range(3):
        pp = _sc_pass(src1d, dst1d, g2d.reshape(_N_PAD), zeros_n)
        g2d, s = _tc_combine(pp[0].reshape(_N_ROWS, _LANES),
                             pp[1].reshape(_N_ROWS, _LANES),
                             g2d, dinv2d, batch2d)
        sums.append(s)

    w_pad = jnp.zeros((8, _LANES), jnp.float32)
    w_pad = w_pad.at[:3, :_CLS].set(jnp.concatenate([W0, W1, W2], axis=0))
    b_pad = jnp.zeros((8, _LANES), jnp.float32)
    b_pad = b_pad.at[:3, :_CLS].set(jnp.stack([b0, b1, b2], axis=0))

    out = _tc_final(sums[0], sums[1], sums[2], counts, w_pad, b_pad)
    return out[:, :_CLS]
